# baseline (speedup 1.0000x reference)
import functools

import jax
import jax.numpy as jnp
from jax import lax
from jax.experimental import pallas as pl
from jax.experimental.pallas import tpu as pltpu
from jax.experimental.pallas import tpu_sc as plsc

N = 5000
C = 128
E = 160000
NC = 2
NS = 16
NW = NC * NS
B = 128
NB = E // B
NBW = (NB + NW - 1) // NW
NPAD = 5120
RPT = NPAD // NS
RB = 1000
ZR = 64
F32 = jnp.float32
I32 = jnp.int32


def _mesh():
    return plsc.VectorSubcoreMesh(
        core_axis_name="c", subcore_axis_name="s", num_cores=NC,
        num_subcores=NS)


def _mm_body(nk, has_v, x_ref, w_ref, *o_refs):
    y = jnp.dot(x_ref[...], w_ref[...], preferred_element_type=F32)
    for j in range(nk):
        o_refs[j][...] = y[:, j * C:(j + 1) * C]
    if has_v:
        o_refs[nk][...] = y[:, nk * C:nk * C + 4]


def _mm(x, ws, vs=None):
    nk = len(ws)
    parts = list(ws)
    if vs is not None:
        parts.append(jnp.pad(vs, ((0, 0), (0, C - 4))))
    wcat = jnp.concatenate(parts, axis=1)
    k_tot = wcat.shape[1]
    out_shape = [jax.ShapeDtypeStruct((N, C), F32) for _ in range(nk)]
    out_specs = [pl.BlockSpec((RB, C), lambda b: (b, 0)) for _ in range(nk)]
    if vs is not None:
        out_shape.append(jax.ShapeDtypeStruct((N, 4), F32))
        out_specs.append(pl.BlockSpec((RB, 4), lambda b: (b, 0)))
    return pl.pallas_call(
        functools.partial(_mm_body, nk, vs is not None),
        grid=(N // RB,),
        in_specs=[pl.BlockSpec((RB, C), lambda b: (b, 0)),
                  pl.BlockSpec((C, k_tot), lambda b: (0, 0))],
        out_specs=out_specs,
        out_shape=out_shape,
    )(x, wcat)


def _inv_body(d_ref, o_ref):
    o_ref[...] = 1.0 / (jnp.sum(d_ref[...], axis=0) + 1e-16)


def _inv_denom(den):
    nwords = den.shape[1]
    return pl.pallas_call(
        _inv_body,
        out_shape=jax.ShapeDtypeStruct((nwords,), F32),
    )(den)


def _gate_body(xz, xr, cz0, cz1, cr0, cr1, h, bz, br, z_ref, hn_ref):
    z = jax.nn.sigmoid(xz[...] + cz0[0, 0] + cz1[0, 0] + bz[...])
    r = jax.nn.sigmoid(xr[...] + cr0[0, 0] + cr1[0, 0] + br[...])
    z_ref[...] = z
    hn_ref[...] = r * h[...]


def _gate(xwz, xwr, cpart, gi_z, gi_r, h, bz, br):
    row = pl.BlockSpec((RB, C), lambda b: (b, 0))
    vec = pl.BlockSpec((1, C), lambda b: (0, 0))

    def cp(gi, c):
        return pl.BlockSpec((1, 1, RB, C), lambda b, gi=gi, c=c: (gi, c, b, 0))

    return pl.pallas_call(
        _gate_body,
        grid=(N // RB,),
        in_specs=[row, row, cp(gi_z, 0), cp(gi_z, 1), cp(gi_r, 0),
                  cp(gi_r, 1), row, vec, vec],
        out_specs=[row, row],
        out_shape=[jax.ShapeDtypeStruct((N, C), F32),
                   jax.ShapeDtypeStruct((N, C), F32)],
    )(xwz, xwr, cpart, cpart, cpart, cpart, h, bz, br)


def _final_body(xn, cn0, cn1, z, h, bn, o_ref):
    n = jnp.tanh(xn[...] + cn0[0, 0] + cn1[0, 0] + bn[...])
    zv = z[...]
    o_ref[...] = (1.0 - zv) * n + zv * h[...]


def _final(xwn, cpart, gi, z, h, bn):
    row = pl.BlockSpec((RB, C), lambda b: (b, 0))
    vec = pl.BlockSpec((1, C), lambda b: (0, 0))

    def cp(c):
        return pl.BlockSpec((1, 1, RB, C), lambda b, c=c: (gi, c, b, 0))

    return pl.pallas_call(
        _final_body,
        grid=(N // RB,),
        in_specs=[row, cp(0), cp(1), row, row, vec],
        out_specs=row,
        out_shape=jax.ShapeDtypeStruct((N, C), F32),
    )(xwn, cpart, cpart, z, h, bn)


def _sc_scalar(ng, su, si, eui, eiu):
    gt = 2 * ng
    scol = (0, 2)[:ng]
    dcol = (1, 3)[:ng]

    def body(su_h, si_h, eui_h, eiu_h, ex_h, den_h,
             su_v, si_v, den_v, src_v, dst_v, ex_v):
        cid = lax.axis_index("c")
        sid = lax.axis_index("s")
        wid = sid * NC + cid
        pltpu.sync_copy(su_h, su_v)
        pltpu.sync_copy(si_h, si_v)
        zero = jnp.zeros((16,), F32)

        def zbody(i, _):
            den_v[pl.ds(i * 16, 16)] = zero
            return 0

        lax.fori_loop(0, gt * NPAD // 16, zbody, 0, unroll=4)

        for et in range(2):
            e_h = eui_h if et == 0 else eiu_h
            tab_s = su_v if et == 0 else si_v
            tab_d = si_v if et == 0 else su_v

            def bbody(k, _, e_h=e_h, tab_s=tab_s, tab_d=tab_d, et=et):
                b = wid + k * NW

                @pl.when(b < NB)
                def _():
                    off = b * B
                    pltpu.sync_copy(e_h.at[0, pl.ds(off, B)], src_v)
                    pltpu.sync_copy(e_h.at[1, pl.ds(off, B)], dst_v)
                    for g in range(ng):
                        gi = et * ng + g
                        for v in range(B // 16):
                            sl = pl.ds(v * 16, 16)
                            s16 = src_v[sl]
                            d16 = dst_v[sl]
                            a_s = plsc.load_gather(
                                tab_s, [s16 * 4 + I32(scol[g])])
                            a_d = plsc.load_gather(
                                tab_d, [d16 * 4 + I32(dcol[g])])
                            al = a_s + a_d
                            al = jnp.where(al >= 0.0, al, al * F32(0.2))
                            exv = jnp.exp(al)
                            ex_v[g, sl] = exv
                            plsc.addupdate_scatter(
                                den_v, [d16 + I32(gi * NPAD)], exv)
                        pltpu.sync_copy(ex_v.at[g],
                                        ex_h.at[gi, pl.ds(off, B)])
                return 0

            lax.fori_loop(0, NBW, bbody, 0)
        pltpu.sync_copy(den_v, den_h.at[wid])

    run = pl.kernel(
        body,
        out_type=(jax.ShapeDtypeStruct((gt, E), F32),
                  jax.ShapeDtypeStruct((NW, gt * NPAD), F32)),
        mesh=_mesh(),
        compiler_params=pltpu.CompilerParams(needs_layout_passes=False),
        scratch_types=[
            pltpu.VMEM((N * 4,), F32),
            pltpu.VMEM((N * 4,), F32),
            pltpu.VMEM((gt * NPAD,), F32),
            pltpu.VMEM((B,), I32),
            pltpu.VMEM((B,), I32),
            pltpu.VMEM((ng, B), F32),
        ],
    )
    return run(su.reshape(-1), si.reshape(-1), eui, eiu)


RD = 3
ID = 4


def _sc_message(ng, xs_list, ex, invd, eui, eiu):
    gt = 2 * ng
    assert len(xs_list) == gt

    def body(*refs):
        xs_hs = refs[0:gt]
        ex_h, invd_h, eui_h, eiu_h = refs[gt:gt + 4]
        cpart_h = refs[gt + 4]
        (invd_v, srcb, dstb, exb, w_v, rows_v, zeros_v, acc,
         isem, gsem, ssem) = refs[gt + 5:]
        cid = lax.axis_index("c")
        sid = lax.axis_index("s")
        wid = sid * NC + cid
        zero = jnp.zeros((16,), F32)

        def zbody(i, _):
            for m in range(C // 16):
                zeros_v[i, pl.ds(m * 16, 16)] = zero
            return 0

        lax.fori_loop(0, ZR, zbody, 0)
        for j in range(RPT // ZR):
            pltpu.sync_copy(zeros_v, acc.at[pl.ds(sid * RPT + j * ZR, ZR)])
        plsc.subcore_barrier()
        nbw = (NB - wid + NW - 1) // NW

        def boff(k):
            return (wid + k * NW) * B

        for gi in range(gt):
            e_h = eui_h if gi < ng else eiu_h
            xs_h = xs_hs[gi]
            pltpu.sync_copy(invd_h.at[pl.ds(gi * NPAD, NPAD)], invd_v)

            def idx_issue(k, slot, e_h=e_h, gi=gi):
                off = boff(k)
                pltpu.async_copy(e_h.at[0, pl.ds(off, B)], srcb.at[slot],
                                 isem.at[slot])
                pltpu.async_copy(e_h.at[1, pl.ds(off, B)], dstb.at[slot],
                                 isem.at[slot])
                pltpu.async_copy(ex_h.at[gi, pl.ds(off, B)],
                                 exb.at[slot], isem.at[slot])

            def idx_wait(k, slot, e_h=e_h, gi=gi):
                off = boff(k)
                pltpu.make_async_copy(e_h.at[0, pl.ds(off, B)],
                                      srcb.at[slot], isem.at[slot]).wait()
                pltpu.make_async_copy(e_h.at[1, pl.ds(off, B)],
                                      dstb.at[slot], isem.at[slot]).wait()
                pltpu.make_async_copy(ex_h.at[gi, pl.ds(off, B)],
                                      exb.at[slot], isem.at[slot]).wait()

            def gather_issue(islot, rslot, xs_h=xs_h):
                pltpu.async_copy(xs_h.at[srcb.at[islot]],
                                 rows_v.at[rslot], gsem.at[rslot])

            def gather_wait(islot, rslot, xs_h=xs_h):
                pltpu.make_async_copy(xs_h.at[srcb.at[islot]],
                                      rows_v.at[rslot],
                                      gsem.at[rslot]).wait()

            def scat_issue(islot, rslot):
                pltpu.async_copy(rows_v.at[rslot], acc.at[dstb.at[islot]],
                                 ssem.at[rslot], add=True)

            def scat_wait(islot, rslot):
                pltpu.make_async_copy(rows_v.at[rslot],
                                      acc.at[dstb.at[islot]],
                                      ssem.at[rslot]).wait()

            idx_issue(0, 0)
            idx_wait(0, 0)
            gather_issue(0, 0)
            idx_issue(1, 1)

            def bbody(k, _):
                islot = lax.rem(k, ID)
                rslot = lax.rem(k, RD)
                nislot = lax.rem(k + 1, ID)
                nrslot = lax.rem(k + 1, RD)

                @pl.when(k >= 2)
                def _():
                    scat_wait(lax.rem(k - 2, ID), lax.rem(k - 2, RD))

                @pl.when(k + 2 < nbw)
                def _():
                    idx_issue(k + 2, lax.rem(k + 2, ID))

                gather_wait(islot, rslot)

                @pl.when(k + 1 < nbw)
                def _():
                    idx_wait(k + 1, nislot)
                    gather_issue(nislot, nrslot)

                for v in range(B // 16):
                    sl = pl.ds(v * 16, 16)
                    d16 = dstb[islot, sl]
                    iv = plsc.load_gather(invd_v, [d16])
                    w_v[sl] = exb[islot, sl] * iv

                @plsc.parallel_loop(0, B, unroll=8)
                def _(i):
                    wb = plsc.load_gather(w_v, [jnp.full((16,), i, I32)])
                    for m in range(C // 16):
                        sl = pl.ds(m * 16, 16)
                        rows_v[rslot, i, sl] = rows_v[rslot, i, sl] * wb

                scat_issue(islot, rslot)
                return 0

            lax.fori_loop(0, nbw, bbody, 0)
            scat_wait(lax.rem(nbw - 2, ID), lax.rem(nbw - 2, RD))
            scat_wait(lax.rem(nbw - 1, ID), lax.rem(nbw - 1, RD))
            plsc.subcore_barrier()
            for j in range(RPT // ZR):
                sl = pl.ds(sid * RPT + j * ZR, ZR)
                pltpu.sync_copy(acc.at[sl], cpart_h.at[gi, cid, sl])
                pltpu.sync_copy(zeros_v, acc.at[sl])
            plsc.subcore_barrier()

    run = pl.kernel(
        body,
        out_type=jax.ShapeDtypeStruct((gt, NC, NPAD, C), F32),
        mesh=_mesh(),
        compiler_params=pltpu.CompilerParams(needs_layout_passes=False),
        scratch_types=[
            pltpu.VMEM((NPAD,), F32),
            pltpu.VMEM((ID, B), I32),
            pltpu.VMEM((ID, B), I32),
            pltpu.VMEM((ID, B), F32),
            pltpu.VMEM((B,), F32),
            pltpu.VMEM((RD, B, C), F32),
            pltpu.VMEM((ZR, C), F32),
            pltpu.VMEM_SHARED((NPAD, C), F32),
            pltpu.SemaphoreType.DMA((ID,)),
            pltpu.SemaphoreType.DMA((RD,)),
            pltpu.SemaphoreType.DMA((RD,)),
        ],
    )
    return run(*xs_list, ex, invd, eui, eiu)


def kernel(x_user, x_item, h_user, h_item, params, edge_index_ui,
           edge_index_iu):
    pz, pr, pn = params['z'], params['r'], params['n']

    def avec(gat, et):
        return (gat[et]['W_src'] @ gat[et]['att_src'],
                gat[et]['W_dst'] @ gat[et]['att_dst'])

    vz_s_ui, vz_d_ui = avec(pz['gat'], 'ui')
    vz_s_iu, vz_d_iu = avec(pz['gat'], 'iu')
    vr_s_ui, vr_d_ui = avec(pr['gat'], 'ui')
    vr_s_iu, vr_d_iu = avec(pr['gat'], 'iu')
    vn_s_ui, vn_d_ui = avec(pn['gat'], 'ui')
    vn_s_iu, vn_d_iu = avec(pn['gat'], 'iu')
    zc = jnp.zeros((C,), F32)
    vu = jnp.stack([vz_s_ui, vz_d_iu, vr_s_ui, vr_d_iu], axis=1)
    vi = jnp.stack([vz_s_iu, vz_d_ui, vr_s_iu, vr_d_ui], axis=1)
    vnu = jnp.stack([vn_s_ui, vn_d_iu, zc, zc], axis=1)
    vni = jnp.stack([vn_s_iu, vn_d_ui, zc, zc], axis=1)

    xs_z_ui, xs_r_ui, su = _mm(
        h_user, [pz['gat']['ui']['W_src'], pr['gat']['ui']['W_src']], vu)
    xs_z_iu, xs_r_iu, si = _mm(
        h_item, [pz['gat']['iu']['W_src'], pr['gat']['iu']['W_src']], vi)
    xwz_u, xwr_u, xwn_u = _mm(
        x_user, [pz['W']['user'], pr['W']['user'], pn['W']['user']])
    xwz_i, xwr_i, xwn_i = _mm(
        x_item, [pz['W']['item'], pr['W']['item'], pn['W']['item']])

    ex_zr, den_zr = _sc_scalar(2, su, si, edge_index_ui, edge_index_iu)
    invd_zr = _inv_denom(den_zr)
    cpart_zr = _sc_message(2, [xs_z_ui, xs_r_ui, xs_z_iu, xs_r_iu],
                           ex_zr, invd_zr, edge_index_ui, edge_index_iu)

    bz_u = pz['gat']['iu']['bias'][None] + pz['b']['user']
    br_u = pr['gat']['iu']['bias'][None] + pr['b']['user']
    bz_i = pz['gat']['ui']['bias'][None] + pz['b']['item']
    br_i = pr['gat']['ui']['bias'][None] + pr['b']['item']
    z_u, hn_u = _gate(xwz_u, xwr_u, cpart_zr, 2, 3, h_user, bz_u, br_u)
    z_i, hn_i = _gate(xwz_i, xwr_i, cpart_zr, 0, 1, h_item, bz_i, br_i)

    xs_n_ui, sn_u = _mm(hn_u, [pn['gat']['ui']['W_src']], vnu)
    xs_n_iu, sn_i = _mm(hn_i, [pn['gat']['iu']['W_src']], vni)

    ex_n, den_n = _sc_scalar(1, sn_u, sn_i, edge_index_ui, edge_index_iu)
    invd_n = _inv_denom(den_n)
    cpart_n = _sc_message(1, [xs_n_ui, xs_n_iu],
                          ex_n, invd_n, edge_index_ui, edge_index_iu)

    bn_u = pn['gat']['iu']['bias'][None] + pn['b']['user']
    bn_i = pn['gat']['ui']['bias'][None] + pn['b']['item']
    new_u = _final(xwn_u, cpart_n, 1, z_u, h_user, bn_u)
    new_i = _final(xwn_i, cpart_n, 0, z_i, h_item, bn_i)
    return (new_u, new_i)


# speedup vs baseline: 1.2762x; 1.2762x over previous
import functools

import jax
import jax.numpy as jnp
from jax import lax
from jax.experimental import pallas as pl
from jax.experimental.pallas import tpu as pltpu
from jax.experimental.pallas import tpu_sc as plsc

N = 5000
C = 128
E = 160000
NC = 2
NS = 16
NW = NC * NS
B = 128
NB = E // B
NBW = (NB + NW - 1) // NW
CH = 1600
NCH = E // CH
NPAD = 5120
RPT = NPAD // NS
RB = 1000
ZR = 64
F32 = jnp.float32
I32 = jnp.int32


def _mesh():
    return plsc.VectorSubcoreMesh(
        core_axis_name="c", subcore_axis_name="s", num_cores=NC,
        num_subcores=NS)


def _mm_body(nk, has_v, x_ref, w_ref, *o_refs):
    y = jnp.dot(x_ref[...], w_ref[...], preferred_element_type=F32)
    for j in range(nk):
        o_refs[j][...] = y[:, j * C:(j + 1) * C]
    if has_v:
        o_refs[nk][...] = y[:, nk * C:nk * C + 4]


def _mm(x, ws, vs=None):
    nk = len(ws)
    parts = list(ws)
    if vs is not None:
        parts.append(jnp.pad(vs, ((0, 0), (0, C - 4))))
    wcat = jnp.concatenate(parts, axis=1)
    k_tot = wcat.shape[1]
    out_shape = [jax.ShapeDtypeStruct((N, C), F32) for _ in range(nk)]
    out_specs = [pl.BlockSpec((RB, C), lambda b: (b, 0)) for _ in range(nk)]
    if vs is not None:
        out_shape.append(jax.ShapeDtypeStruct((N, 4), F32))
        out_specs.append(pl.BlockSpec((RB, 4), lambda b: (b, 0)))
    return pl.pallas_call(
        functools.partial(_mm_body, nk, vs is not None),
        grid=(N // RB,),
        in_specs=[pl.BlockSpec((RB, C), lambda b: (b, 0)),
                  pl.BlockSpec((C, k_tot), lambda b: (0, 0))],
        out_specs=out_specs,
        out_shape=out_shape,
    )(x, wcat)


def _inv_body(d_ref, o_ref):
    o_ref[...] = 1.0 / (jnp.sum(d_ref[...], axis=0) + 1e-16)


def _inv_denom(den):
    nwords = den.shape[1]
    return pl.pallas_call(
        _inv_body,
        out_shape=jax.ShapeDtypeStruct((nwords,), F32),
    )(den)


def _gate_body(xz, xr, cz0, cz1, cr0, cr1, h, bz, br, z_ref, hn_ref):
    z = jax.nn.sigmoid(xz[...] + cz0[0, 0] + cz1[0, 0] + bz[...])
    r = jax.nn.sigmoid(xr[...] + cr0[0, 0] + cr1[0, 0] + br[...])
    z_ref[...] = z
    hn_ref[...] = r * h[...]


def _gate(xwz, xwr, cpart, gi_z, gi_r, h, bz, br):
    row = pl.BlockSpec((RB, C), lambda b: (b, 0))
    vec = pl.BlockSpec((1, C), lambda b: (0, 0))

    def cp(gi, c):
        return pl.BlockSpec((1, 1, RB, C), lambda b, gi=gi, c=c: (gi, c, b, 0))

    return pl.pallas_call(
        _gate_body,
        grid=(N // RB,),
        in_specs=[row, row, cp(gi_z, 0), cp(gi_z, 1), cp(gi_r, 0),
                  cp(gi_r, 1), row, vec, vec],
        out_specs=[row, row],
        out_shape=[jax.ShapeDtypeStruct((N, C), F32),
                   jax.ShapeDtypeStruct((N, C), F32)],
    )(xwz, xwr, cpart, cpart, cpart, cpart, h, bz, br)


def _final_body(xn, cn0, cn1, z, h, bn, o_ref):
    n = jnp.tanh(xn[...] + cn0[0, 0] + cn1[0, 0] + bn[...])
    zv = z[...]
    o_ref[...] = (1.0 - zv) * n + zv * h[...]


def _final(xwn, cpart, gi, z, h, bn):
    row = pl.BlockSpec((RB, C), lambda b: (b, 0))
    vec = pl.BlockSpec((1, C), lambda b: (0, 0))

    def cp(c):
        return pl.BlockSpec((1, 1, RB, C), lambda b, c=c: (gi, c, b, 0))

    return pl.pallas_call(
        _final_body,
        grid=(N // RB,),
        in_specs=[row, cp(0), cp(1), row, row, vec],
        out_specs=row,
        out_shape=jax.ShapeDtypeStruct((N, C), F32),
    )(xwn, cpart, cpart, z, h, bn)


def _sc_scalar(ng, su, si, eui, eiu):
    gt = 2 * ng
    scol = (0, 2)[:ng]
    dcol = (1, 3)[:ng]

    def body(su_h, si_h, eui_h, eiu_h, ex_h, den_h,
             su_v, si_v, den_v, src_v, dst_v, ex_v):
        cid = lax.axis_index("c")
        sid = lax.axis_index("s")
        wid = sid * NC + cid
        pltpu.sync_copy(su_h, su_v)
        pltpu.sync_copy(si_h, si_v)
        zero = jnp.zeros((16,), F32)

        def zbody(i, _):
            den_v[pl.ds(i * 16, 16)] = zero
            return 0

        lax.fori_loop(0, gt * NPAD // 16, zbody, 0, unroll=4)

        for et in range(2):
            e_h = eui_h if et == 0 else eiu_h
            tab_s = su_v if et == 0 else si_v
            tab_d = si_v if et == 0 else su_v

            def cbody(j, _, e_h=e_h, tab_s=tab_s, tab_d=tab_d, et=et):
                off = (wid + j * NW) * CH
                pltpu.sync_copy(e_h.at[pl.ds(off, CH)], src_v)
                pltpu.sync_copy(e_h.at[pl.ds(E + off, CH)], dst_v)
                for g in range(ng):
                    gi = et * ng + g

                    @plsc.parallel_loop(0, CH // 16, unroll=4)
                    def _(v, g=g, gi=gi, tab_s=tab_s, tab_d=tab_d):
                        sl = pl.ds(v * 16, 16)
                        s16 = src_v[sl]
                        d16 = dst_v[sl]
                        a_s = plsc.load_gather(
                            tab_s, [s16 * 4 + I32(scol[g])])
                        a_d = plsc.load_gather(
                            tab_d, [d16 * 4 + I32(dcol[g])])
                        al = a_s + a_d
                        al = jnp.where(al >= 0.0, al, al * F32(0.2))
                        exv = jnp.exp(al)
                        ex_v[pl.ds(g * CH + v * 16, 16)] = exv
                        plsc.addupdate_scatter(
                            den_v, [d16 + I32(gi * NPAD)], exv)

                    pltpu.sync_copy(ex_v.at[pl.ds(g * CH, CH)],
                                    ex_h.at[pl.ds(gi * E + off, CH)])
                return 0

            nchw = (NCH - wid + NW - 1) // NW
            lax.fori_loop(0, nchw, cbody, 0)
        pltpu.sync_copy(den_v, den_h.at[wid])

    run = pl.kernel(
        body,
        out_type=(jax.ShapeDtypeStruct((gt * E,), F32),
                  jax.ShapeDtypeStruct((NW, gt * NPAD), F32)),
        mesh=_mesh(),
        compiler_params=pltpu.CompilerParams(needs_layout_passes=False),
        scratch_types=[
            pltpu.VMEM((N * 4,), F32),
            pltpu.VMEM((N * 4,), F32),
            pltpu.VMEM((gt * NPAD,), F32),
            pltpu.VMEM((CH,), I32),
            pltpu.VMEM((CH,), I32),
            pltpu.VMEM((ng * CH,), F32),
        ],
    )
    return run(su.reshape(-1), si.reshape(-1), eui.reshape(-1),
               eiu.reshape(-1))


RD = 3
ID = 4


def _sc_message(ng, xs_list, ex, invd, eui, eiu):
    gt = 2 * ng
    assert len(xs_list) == gt

    def body(*refs):
        xs_hs = refs[0:gt]
        ex_h, invd_h, eui_h, eiu_h = refs[gt:gt + 4]
        cpart_h = refs[gt + 4]
        (invd_v, srcb, dstb, exb, w_v, rows_v, zeros_v, acc,
         isem, gsem, ssem) = refs[gt + 5:]
        cid = lax.axis_index("c")
        sid = lax.axis_index("s")
        wid = sid * NC + cid
        zero = jnp.zeros((16,), F32)

        def zbody(i, _):
            for m in range(C // 16):
                zeros_v[i, pl.ds(m * 16, 16)] = zero
            return 0

        lax.fori_loop(0, ZR, zbody, 0)
        for j in range(RPT // ZR):
            pltpu.sync_copy(zeros_v, acc.at[pl.ds(sid * RPT + j * ZR, ZR)])
        plsc.subcore_barrier()
        nbw = (NB - wid + NW - 1) // NW

        def boff(k):
            return (wid + k * NW) * B

        for gi in range(gt):
            e_h = eui_h if gi < ng else eiu_h
            xs_h = xs_hs[gi]
            pltpu.sync_copy(invd_h.at[pl.ds(gi * NPAD, NPAD)], invd_v)

            def idx_issue(k, slot, e_h=e_h, gi=gi):
                off = boff(k)
                pltpu.async_copy(e_h.at[pl.ds(off, B)], srcb.at[slot],
                                 isem.at[slot])
                pltpu.async_copy(e_h.at[pl.ds(E + off, B)], dstb.at[slot],
                                 isem.at[slot])
                pltpu.async_copy(ex_h.at[pl.ds(gi * E + off, B)],
                                 exb.at[slot], isem.at[slot])

            def idx_wait(k, slot, e_h=e_h, gi=gi):
                off = boff(k)
                pltpu.make_async_copy(e_h.at[pl.ds(off, B)],
                                      srcb.at[slot], isem.at[slot]).wait()
                pltpu.make_async_copy(e_h.at[pl.ds(E + off, B)],
                                      dstb.at[slot], isem.at[slot]).wait()
                pltpu.make_async_copy(ex_h.at[pl.ds(gi * E + off, B)],
                                      exb.at[slot], isem.at[slot]).wait()

            def gather_issue(islot, rslot, xs_h=xs_h):
                pltpu.async_copy(xs_h.at[srcb.at[islot]],
                                 rows_v.at[rslot], gsem.at[rslot])

            def gather_wait(islot, rslot, xs_h=xs_h):
                pltpu.make_async_copy(xs_h.at[srcb.at[islot]],
                                      rows_v.at[rslot],
                                      gsem.at[rslot]).wait()

            def scat_issue(islot, rslot):
                pltpu.async_copy(rows_v.at[rslot], acc.at[dstb.at[islot]],
                                 ssem.at[rslot], add=True)

            def scat_wait(islot, rslot):
                pltpu.make_async_copy(rows_v.at[rslot],
                                      acc.at[dstb.at[islot]],
                                      ssem.at[rslot]).wait()

            idx_issue(0, 0)
            idx_wait(0, 0)
            gather_issue(0, 0)
            idx_issue(1, 1)

            def bbody(k, _):
                islot = lax.rem(k, ID)
                rslot = lax.rem(k, RD)
                nislot = lax.rem(k + 1, ID)
                nrslot = lax.rem(k + 1, RD)

                @pl.when(k >= 2)
                def _():
                    scat_wait(lax.rem(k - 2, ID), lax.rem(k - 2, RD))

                @pl.when(k + 2 < nbw)
                def _():
                    idx_issue(k + 2, lax.rem(k + 2, ID))

                gather_wait(islot, rslot)

                @pl.when(k + 1 < nbw)
                def _():
                    idx_wait(k + 1, nislot)
                    gather_issue(nislot, nrslot)

                for v in range(B // 16):
                    sl = pl.ds(v * 16, 16)
                    d16 = dstb[islot, sl]
                    iv = plsc.load_gather(invd_v, [d16])
                    w_v[sl] = exb[islot, sl] * iv

                @plsc.parallel_loop(0, B, unroll=8)
                def _(i):
                    wb = plsc.load_gather(w_v, [jnp.full((16,), i, I32)])
                    for m in range(C // 16):
                        sl = pl.ds(m * 16, 16)
                        rows_v[rslot, i, sl] = rows_v[rslot, i, sl] * wb

                scat_issue(islot, rslot)
                return 0

            lax.fori_loop(0, nbw, bbody, 0)
            scat_wait(lax.rem(nbw - 2, ID), lax.rem(nbw - 2, RD))
            scat_wait(lax.rem(nbw - 1, ID), lax.rem(nbw - 1, RD))
            plsc.subcore_barrier()
            for j in range(RPT // ZR):
                sl = pl.ds(sid * RPT + j * ZR, ZR)
                pltpu.sync_copy(acc.at[sl], cpart_h.at[gi, cid, sl])
                pltpu.sync_copy(zeros_v, acc.at[sl])
            plsc.subcore_barrier()

    run = pl.kernel(
        body,
        out_type=jax.ShapeDtypeStruct((gt, NC, NPAD, C), F32),
        mesh=_mesh(),
        compiler_params=pltpu.CompilerParams(needs_layout_passes=False),
        scratch_types=[
            pltpu.VMEM((NPAD,), F32),
            pltpu.VMEM((ID, B), I32),
            pltpu.VMEM((ID, B), I32),
            pltpu.VMEM((ID, B), F32),
            pltpu.VMEM((B,), F32),
            pltpu.VMEM((RD, B, C), F32),
            pltpu.VMEM((ZR, C), F32),
            pltpu.VMEM_SHARED((NPAD, C), F32),
            pltpu.SemaphoreType.DMA((ID,)),
            pltpu.SemaphoreType.DMA((RD,)),
            pltpu.SemaphoreType.DMA((RD,)),
        ],
    )
    return run(*xs_list, ex, invd, eui.reshape(-1), eiu.reshape(-1))


def kernel(x_user, x_item, h_user, h_item, params, edge_index_ui,
           edge_index_iu):
    pz, pr, pn = params['z'], params['r'], params['n']

    def avec(gat, et):
        return (gat[et]['W_src'] @ gat[et]['att_src'],
                gat[et]['W_dst'] @ gat[et]['att_dst'])

    vz_s_ui, vz_d_ui = avec(pz['gat'], 'ui')
    vz_s_iu, vz_d_iu = avec(pz['gat'], 'iu')
    vr_s_ui, vr_d_ui = avec(pr['gat'], 'ui')
    vr_s_iu, vr_d_iu = avec(pr['gat'], 'iu')
    vn_s_ui, vn_d_ui = avec(pn['gat'], 'ui')
    vn_s_iu, vn_d_iu = avec(pn['gat'], 'iu')
    zc = jnp.zeros((C,), F32)
    vu = jnp.stack([vz_s_ui, vz_d_iu, vr_s_ui, vr_d_iu], axis=1)
    vi = jnp.stack([vz_s_iu, vz_d_ui, vr_s_iu, vr_d_ui], axis=1)
    vnu = jnp.stack([vn_s_ui, vn_d_iu, zc, zc], axis=1)
    vni = jnp.stack([vn_s_iu, vn_d_ui, zc, zc], axis=1)

    xs_z_ui, xs_r_ui, su = _mm(
        h_user, [pz['gat']['ui']['W_src'], pr['gat']['ui']['W_src']], vu)
    xs_z_iu, xs_r_iu, si = _mm(
        h_item, [pz['gat']['iu']['W_src'], pr['gat']['iu']['W_src']], vi)
    xwz_u, xwr_u, xwn_u = _mm(
        x_user, [pz['W']['user'], pr['W']['user'], pn['W']['user']])
    xwz_i, xwr_i, xwn_i = _mm(
        x_item, [pz['W']['item'], pr['W']['item'], pn['W']['item']])

    ex_zr, den_zr = _sc_scalar(2, su, si, edge_index_ui, edge_index_iu)
    invd_zr = _inv_denom(den_zr)
    cpart_zr = _sc_message(2, [xs_z_ui, xs_r_ui, xs_z_iu, xs_r_iu],
                           ex_zr, invd_zr, edge_index_ui, edge_index_iu)

    bz_u = pz['gat']['iu']['bias'][None] + pz['b']['user']
    br_u = pr['gat']['iu']['bias'][None] + pr['b']['user']
    bz_i = pz['gat']['ui']['bias'][None] + pz['b']['item']
    br_i = pr['gat']['ui']['bias'][None] + pr['b']['item']
    z_u, hn_u = _gate(xwz_u, xwr_u, cpart_zr, 2, 3, h_user, bz_u, br_u)
    z_i, hn_i = _gate(xwz_i, xwr_i, cpart_zr, 0, 1, h_item, bz_i, br_i)

    xs_n_ui, sn_u = _mm(hn_u, [pn['gat']['ui']['W_src']], vnu)
    xs_n_iu, sn_i = _mm(hn_i, [pn['gat']['iu']['W_src']], vni)

    ex_n, den_n = _sc_scalar(1, sn_u, sn_i, edge_index_ui, edge_index_iu)
    invd_n = _inv_denom(den_n)
    cpart_n = _sc_message(1, [xs_n_ui, xs_n_iu],
                          ex_n, invd_n, edge_index_ui, edge_index_iu)

    bn_u = pn['gat']['iu']['bias'][None] + pn['b']['user']
    bn_i = pn['gat']['ui']['bias'][None] + pn['b']['item']
    new_u = _final(xwn_u, cpart_n, 1, z_u, h_user, bn_u)
    new_i = _final(xwn_i, cpart_n, 0, z_i, h_item, bn_i)
    return (new_u, new_i)
